# baseline (speedup 1.0000x reference)
import functools

import jax
import jax.numpy as jnp
from jax import lax
from jax.experimental import pallas as pl
from jax.experimental.pallas import tpu as pltpu
from jax.experimental.pallas import tpu_sc as plsc

N = 10000
D = 128
H = 128
C = 64
B = 1024
N1 = 4096
E0 = 131072
E1 = 32768
TEMP = 1.0

NCORES = 2
NSUB = 16
NW = NCORES * NSUB
EBLK = 128
CA = 16
CB = 128


def _make_sc_agg(n_tab, n_blk):
    rpt = B // NSUB
    mesh = plsc.VectorSubcoreMesh(core_axis_name="c", subcore_axis_name="s")

    @functools.partial(
        pl.kernel,
        out_type=jax.ShapeDtypeStruct((NCORES, B, D), jnp.float32),
        mesh=mesh,
        scratch_types=(
            pltpu.VMEM((n_blk, EBLK), jnp.int32),
            pltpu.VMEM((n_blk, EBLK), jnp.int32),
            pltpu.VMEM((4, EBLK, D), jnp.float32),
            pltpu.VMEM_SHARED((n_tab, D), jnp.float32),
            tuple(pltpu.SemaphoreType.DMA for _ in range(4)),
            tuple(pltpu.SemaphoreType.DMA for _ in range(4)),
        ),
    )
    def agg(table, src, dst, zrows,
            accum_out,
            src_v, dst_v, rows_v, accum_sh, gsem, ssem):
        cid = lax.axis_index("c")
        sid = lax.axis_index("s")
        wid = sid * NCORES + cid
        r0 = sid * rpt
        pltpu.sync_copy(src.at[wid], src_v)
        pltpu.sync_copy(dst.at[wid], dst_v)
        for p in range(4):
            pltpu.async_copy(table.at[src_v.at[p]], rows_v.at[p], gsem[p])
        pltpu.sync_copy(zrows.at[pl.ds(r0, rpt)], accum_sh.at[pl.ds(r0, rpt)])
        plsc.subcore_barrier()

        def body(jj, carry):
            for p in range(4):
                j = 4 * jj + p
                pltpu.make_async_copy(
                    table.at[src_v.at[j]], rows_v.at[p], gsem[p]).wait()
                pltpu.async_copy(
                    rows_v.at[p], accum_sh.at[dst_v.at[j]], ssem[p], add=True)
            for p in range(4):
                j = 4 * jj + p
                pltpu.make_async_copy(
                    rows_v.at[p], accum_sh.at[dst_v.at[j]], ssem[p]).wait()

                @pl.when(j + 4 < n_blk)
                def _():
                    pltpu.async_copy(
                        table.at[src_v.at[j + 4]], rows_v.at[p], gsem[p])
            return carry

        lax.fori_loop(0, n_blk // 4, body, 0)
        plsc.subcore_barrier()
        pltpu.sync_copy(accum_sh.at[pl.ds(r0, rpt)],
                        accum_out.at[cid, pl.ds(r0, rpt)])

    return agg


_sc_agg0 = _make_sc_agg(N1, E0 // NW // EBLK)
_sc_agg1 = _make_sc_agg(B, E1 // NW // EBLK)


def _make_tc_cnt(rows):
    def _tc_cnt(dst_ref, out_ref):
        def body(r, acc):
            d = jnp.minimum(dst_ref[r, :], B)
            a = d >> 7
            b = d & 127
            aet = (lax.broadcasted_iota(jnp.int32, (CA, 512), 0)
                   == a[None, :]).astype(jnp.bfloat16)
            be = (b[:, None] == lax.broadcasted_iota(
                jnp.int32, (512, CB), 1)).astype(jnp.bfloat16)
            return acc + lax.dot_general(
                aet, be, (((1,), (0,)), ((), ())),
                preferred_element_type=jnp.float32)

        out_ref[...] = lax.fori_loop(
            0, rows, body, jnp.zeros((CA, CB), jnp.float32))

    return _tc_cnt


def _tc_cnt_call(dst_flat, n_edges):
    rows = n_edges // 512
    return pl.pallas_call(
        _make_tc_cnt(rows),
        out_shape=jax.ShapeDtypeStruct((CA, CB), jnp.float32),
    )(dst_flat.reshape(rows, 512))


def _tc_sage(acc_ref, cnt_ref, xt_ref, wl_ref, bl_ref, wr_ref, out_ref):
    acc = acc_ref[0] + acc_ref[1]
    agg = acc / jnp.clip(cnt_ref[...], 1.0)
    h = (jnp.dot(agg, wl_ref[...], preferred_element_type=jnp.float32)
         + bl_ref[0][None, :]
         + jnp.dot(xt_ref[...], wr_ref[...], preferred_element_type=jnp.float32))
    out_ref[...] = jnp.maximum(h, 0.0)


def _tc_sage_call(acc, cnt_col, xt, wl, bl, wr):
    blk = 512
    return pl.pallas_call(
        _tc_sage,
        grid=(B // blk,),
        in_specs=[
            pl.BlockSpec((NCORES, blk, D), lambda i: (0, i, 0)),
            pl.BlockSpec((blk, 1), lambda i: (i, 0)),
            pl.BlockSpec((blk, D), lambda i: (i, 0)),
            pl.BlockSpec((D, H), lambda i: (0, 0)),
            pl.BlockSpec((1, H), lambda i: (0, 0)),
            pl.BlockSpec((D, H), lambda i: (0, 0)),
        ],
        out_specs=pl.BlockSpec((blk, H), lambda i: (i, 0)),
        out_shape=jax.ShapeDtypeStruct((B, H), jnp.float32),
    )(acc, cnt_col, xt, wl, bl.reshape(1, H), wr)


def _tc_final(acc2_ref, cnt2_ref, h1_ref, xt_ref,
              w2l_ref, b2l_ref, w2r_ref, w1r_ref, wout_ref, bout_ref,
              out_ref):
    acc = acc2_ref[0] + acc2_ref[1]
    agg = acc / jnp.clip(cnt2_ref[...], 1.0)
    h1 = h1_ref[...]
    h2 = (jnp.dot(agg, w2l_ref[...], preferred_element_type=jnp.float32)
          + b2l_ref[0][None, :]
          + jnp.dot(h1, w2r_ref[...], preferred_element_type=jnp.float32))
    h2 = jnp.maximum(h2, 0.0)
    ht1 = jnp.maximum(
        jnp.dot(xt_ref[...], w1r_ref[...], preferred_element_type=jnp.float32), 0.0)
    ht2 = jnp.maximum(
        jnp.dot(ht1, w2r_ref[...], preferred_element_type=jnp.float32), 0.0)
    scale = TEMP * jnp.sqrt(jnp.float32(H))
    s0 = jnp.sum(h1 * ht1, axis=-1) / scale
    s1 = jnp.sum(h2 * ht2, axis=-1) / scale
    m = jnp.maximum(s0, s1)
    e0 = jnp.exp(s0 - m)
    e1 = jnp.exp(s1 - m)
    a0 = (e0 / (e0 + e1))[:, None]
    hsum = a0 * h1 + (1.0 - a0) * h2
    out_ref[...] = (jnp.dot(hsum, wout_ref[...], preferred_element_type=jnp.float32)
                    + bout_ref[0][None, :])


def _tc_final_call(acc2, cnt2_col, h1b, xt, w2l, b2l, w2r, w1r, wout, bout):
    return pl.pallas_call(
        _tc_final,
        out_shape=jax.ShapeDtypeStruct((B, C), jnp.float32),
    )(acc2, cnt2_col, h1b, xt, w2l, b2l.reshape(1, H), w2r, w1r,
      wout, bout.reshape(1, C))


def kernel(x, edge_index0, edge_index1, batch_size,
           W1l, b1l, W1r, W2l, b2l, W2r, Wout, bout):
    nb0 = E0 // NW // EBLK
    nb1 = E1 // NW // EBLK
    src0 = edge_index0[0].reshape(NW, nb0, EBLK)
    dst0 = edge_index0[1].reshape(NW, nb0, EBLK)
    src1 = edge_index1[0].reshape(NW, nb1, EBLK)
    dst1 = edge_index1[1].reshape(NW, nb1, EBLK)

    zrows = jnp.zeros((B, 128), jnp.float32)

    cnt0_col = _tc_cnt_call(edge_index0[1], E0).reshape(-1)[:B, None]
    cnt1_col = _tc_cnt_call(edge_index1[1], E1).reshape(-1)[:B, None]

    acc0 = _sc_agg0(x, src0, dst0, zrows)
    h1b = _tc_sage_call(acc0, cnt0_col, x[:B], W1l, b1l, W1r)

    acc1 = _sc_agg1(h1b, src1, dst1, zrows)

    x_ = lax.dynamic_slice_in_dim(x, batch_size - B, B, axis=0)
    return _tc_final_call(acc1, cnt1_col, h1b, x_, W2l, b2l, W2r, W1r, Wout, bout)


# speedup vs baseline: 1.3812x; 1.3812x over previous
import functools

import jax
import jax.numpy as jnp
from jax import lax
from jax.experimental import pallas as pl
from jax.experimental.pallas import tpu as pltpu
from jax.experimental.pallas import tpu_sc as plsc

N = 10000
D = 128
H = 128
C = 64
B = 1024
N1 = 4096
E0 = 131072
E1 = 32768
TEMP = 1.0

NCORES = 2
NSUB = 16
NW = NCORES * NSUB
EBLK = 128
CA = 40
CB = 32


def _make_sc_agg(n_tab, n_blk):
    rpt = B // NSUB
    mesh = plsc.VectorSubcoreMesh(core_axis_name="c", subcore_axis_name="s")

    @functools.partial(
        pl.kernel,
        out_type=jax.ShapeDtypeStruct((NCORES, B, D), jnp.float32),
        mesh=mesh,
        scratch_types=(
            pltpu.VMEM((n_blk, EBLK), jnp.int32),
            pltpu.VMEM((n_blk, EBLK), jnp.int32),
            pltpu.VMEM((4, EBLK, D), jnp.float32),
            pltpu.VMEM_SHARED((n_tab, D), jnp.float32),
            tuple(pltpu.SemaphoreType.DMA for _ in range(4)),
            tuple(pltpu.SemaphoreType.DMA for _ in range(4)),
        ),
    )
    def agg(table, src, dst, zrows,
            accum_out,
            src_v, dst_v, rows_v, accum_sh, gsem, ssem):
        cid = lax.axis_index("c")
        sid = lax.axis_index("s")
        wid = sid * NCORES + cid
        r0 = sid * rpt
        pltpu.sync_copy(src.at[wid], src_v)
        pltpu.sync_copy(dst.at[wid], dst_v)
        for p in range(4):
            pltpu.async_copy(table.at[src_v.at[p]], rows_v.at[p], gsem[p])
        pltpu.sync_copy(zrows.at[pl.ds(r0, rpt)], accum_sh.at[pl.ds(r0, rpt)])
        plsc.subcore_barrier()

        def body(jj, carry):
            for p in range(4):
                j = 4 * jj + p
                pltpu.make_async_copy(
                    table.at[src_v.at[j]], rows_v.at[p], gsem[p]).wait()
                pltpu.async_copy(
                    rows_v.at[p], accum_sh.at[dst_v.at[j]], ssem[p], add=True)
            for p in range(4):
                j = 4 * jj + p
                pltpu.make_async_copy(
                    rows_v.at[p], accum_sh.at[dst_v.at[j]], ssem[p]).wait()

                @pl.when(j + 4 < n_blk)
                def _():
                    pltpu.async_copy(
                        table.at[src_v.at[j + 4]], rows_v.at[p], gsem[p])
            return carry

        lax.fori_loop(0, n_blk // 4, body, 0)
        plsc.subcore_barrier()
        pltpu.sync_copy(accum_sh.at[pl.ds(r0, rpt)],
                        accum_out.at[cid, pl.ds(r0, rpt)])

    return agg


_sc_agg0 = _make_sc_agg(N1, E0 // NW // EBLK)
_sc_agg1 = _make_sc_agg(B, E1 // NW // EBLK)


def _tc_cnt(dst_ref, out_ref):
    @pl.when(pl.program_id(0) == 0)
    def _():
        out_ref[...] = jnp.zeros_like(out_ref)

    partial = jnp.zeros((CA, CB), jnp.float32)
    for r in range(64):
        d = jnp.minimum(dst_ref[r, :], B)
        a = d >> 5
        b = d & 31
        aet = (lax.broadcasted_iota(jnp.int32, (CA, 512), 0)
               == a[None, :]).astype(jnp.bfloat16)
        be = (b[:, None] == lax.broadcasted_iota(
            jnp.int32, (512, CB), 1)).astype(jnp.bfloat16)
        partial += lax.dot_general(
            aet, be, (((1,), (0,)), ((), ())),
            preferred_element_type=jnp.float32)
    out_ref[...] += partial


def _tc_cnt_call(dst_flat, n_edges):
    rows = n_edges // 512
    return pl.pallas_call(
        _tc_cnt,
        grid=(rows // 64,),
        in_specs=[pl.BlockSpec((64, 512), lambda i: (i, 0))],
        out_specs=pl.BlockSpec((CA, CB), lambda i: (0, 0)),
        out_shape=jax.ShapeDtypeStruct((CA, CB), jnp.float32),
    )(dst_flat.reshape(rows, 512))


def _tc_sage(acc_ref, cnt_ref, xt_ref, wl_ref, bl_ref, wr_ref, out_ref):
    acc = acc_ref[0] + acc_ref[1]
    agg = acc / jnp.clip(cnt_ref[...], 1.0)
    h = (jnp.dot(agg, wl_ref[...], preferred_element_type=jnp.float32)
         + bl_ref[0][None, :]
         + jnp.dot(xt_ref[...], wr_ref[...], preferred_element_type=jnp.float32))
    out_ref[...] = jnp.maximum(h, 0.0)


def _tc_sage_call(acc, cnt_col, xt, wl, bl, wr):
    blk = 512
    return pl.pallas_call(
        _tc_sage,
        grid=(B // blk,),
        in_specs=[
            pl.BlockSpec((NCORES, blk, D), lambda i: (0, i, 0)),
            pl.BlockSpec((blk, 1), lambda i: (i, 0)),
            pl.BlockSpec((blk, D), lambda i: (i, 0)),
            pl.BlockSpec((D, H), lambda i: (0, 0)),
            pl.BlockSpec((1, H), lambda i: (0, 0)),
            pl.BlockSpec((D, H), lambda i: (0, 0)),
        ],
        out_specs=pl.BlockSpec((blk, H), lambda i: (i, 0)),
        out_shape=jax.ShapeDtypeStruct((B, H), jnp.float32),
    )(acc, cnt_col, xt, wl, bl.reshape(1, H), wr)


def _tc_final(acc2_ref, cnt2_ref, h1_ref, xt_ref,
              w2l_ref, b2l_ref, w2r_ref, w1r_ref, wout_ref, bout_ref,
              out_ref):
    acc = acc2_ref[0] + acc2_ref[1]
    agg = acc / jnp.clip(cnt2_ref[...], 1.0)
    h1 = h1_ref[...]
    h2 = (jnp.dot(agg, w2l_ref[...], preferred_element_type=jnp.float32)
          + b2l_ref[0][None, :]
          + jnp.dot(h1, w2r_ref[...], preferred_element_type=jnp.float32))
    h2 = jnp.maximum(h2, 0.0)
    ht1 = jnp.maximum(
        jnp.dot(xt_ref[...], w1r_ref[...], preferred_element_type=jnp.float32), 0.0)
    ht2 = jnp.maximum(
        jnp.dot(ht1, w2r_ref[...], preferred_element_type=jnp.float32), 0.0)
    scale = TEMP * jnp.sqrt(jnp.float32(H))
    s0 = jnp.sum(h1 * ht1, axis=-1) / scale
    s1 = jnp.sum(h2 * ht2, axis=-1) / scale
    m = jnp.maximum(s0, s1)
    e0 = jnp.exp(s0 - m)
    e1 = jnp.exp(s1 - m)
    a0 = (e0 / (e0 + e1))[:, None]
    hsum = a0 * h1 + (1.0 - a0) * h2
    out_ref[...] = (jnp.dot(hsum, wout_ref[...], preferred_element_type=jnp.float32)
                    + bout_ref[0][None, :])


def _tc_final_call(acc2, cnt2_col, h1b, xt, w2l, b2l, w2r, w1r, wout, bout):
    return pl.pallas_call(
        _tc_final,
        out_shape=jax.ShapeDtypeStruct((B, C), jnp.float32),
    )(acc2, cnt2_col, h1b, xt, w2l, b2l.reshape(1, H), w2r, w1r,
      wout, bout.reshape(1, C))


def kernel(x, edge_index0, edge_index1, batch_size,
           W1l, b1l, W1r, W2l, b2l, W2r, Wout, bout):
    nb0 = E0 // NW // EBLK
    nb1 = E1 // NW // EBLK
    src0 = edge_index0[0].reshape(NW, nb0, EBLK)
    dst0 = edge_index0[1].reshape(NW, nb0, EBLK)
    src1 = edge_index1[0].reshape(NW, nb1, EBLK)
    dst1 = edge_index1[1].reshape(NW, nb1, EBLK)

    zrows = jnp.zeros((B, 128), jnp.float32)

    cnt0_col = _tc_cnt_call(edge_index0[1], E0).reshape(-1)[:B, None]
    cnt1_col = _tc_cnt_call(edge_index1[1], E1).reshape(-1)[:B, None]

    acc0 = _sc_agg0(x, src0, dst0, zrows)
    h1b = _tc_sage_call(acc0, cnt0_col, x[:B], W1l, b1l, W1r)

    acc1 = _sc_agg1(h1b, src1, dst1, zrows)

    x_ = lax.dynamic_slice_in_dim(x, batch_size - B, B, axis=0)
    return _tc_final_call(acc1, cnt1_col, h1b, x_, W2l, b2l, W2r, W1r, Wout, bout)
